# baseline (speedup 1.0000x reference)
import functools

import jax
import jax.numpy as jnp
from jax import lax
from jax.experimental import pallas as pl
from jax.experimental.pallas import tpu as pltpu
from jax.experimental.pallas import tpu_sc as plsc

B = 8
NS = 1250
ES = 20000
STRIDE = 1280
RP = B * STRIDE
RM = 2 * RP
EP = 20480
F32 = jnp.float32


def _mm(a, b):
    return jnp.dot(a, b, preferred_element_type=F32)


def _pad_nodes(nodes):
    G = nodes.shape[0]
    pad = jnp.broadcast_to(nodes[:, -1:], (G, STRIDE - NS))
    return jnp.concatenate([nodes, pad], axis=1).reshape(-1)


def _edge_idx(ei, S):
    G = ei.shape[0]
    g = jnp.arange(G, dtype=jnp.int32)[:, None]
    src = jnp.concatenate([ei[:, 0, :], jnp.zeros((G, EP - ES), jnp.int32)], axis=1)
    dst = jnp.concatenate([ei[:, 1, :], jnp.full((G, EP - ES), NS, jnp.int32)], axis=1)
    srcg = src + g * STRIDE
    dstl = dst + (g % S) * STRIDE
    nch = EP * S // 16 // 128
    return srcg.reshape(2, 16, nch, 128), dstl.reshape(2, 16, nch, 128)


def _deg_idx(eis):
    parts = []
    for k, ei in enumerate(eis):
        G = ei.shape[0]
        g = jnp.arange(G, dtype=jnp.int32)[:, None]
        d = (ei[:, 1, :] + g * STRIDE).reshape(-1)
        d = jnp.concatenate([d, jnp.full((G * EP - G * ES,), NS, jnp.int32)])
        parts.append(d + k * RP)
    return jnp.stack(parts).reshape(3, 2, 16, 40, 128)


@functools.lru_cache(maxsize=None)
def _mesh():
    return plsc.VectorSubcoreMesh(core_axis_name="c", subcore_axis_name="s")


@functools.lru_cache(maxsize=None)
def _get_gather_deg():
    return functools.partial(
        pl.kernel, mesh=_mesh(),
        out_type=[jax.ShapeDtypeStruct((RP, 128), F32),
                  jax.ShapeDtypeStruct((RP, 128), F32),
                  jax.ShapeDtypeStruct((RM, 128), F32),
                  jax.ShapeDtypeStruct((2, 3 * RP), F32)],
        scratch_types=[pltpu.VMEM((4, 80), jnp.int32),
                       pltpu.VMEM((8, 80), jnp.int32),
                       pltpu.VMEM((40, 128), jnp.int32),
                       pltpu.VMEM((640, 128), F32),
                       pltpu.VMEM((128,), F32),
                       pltpu.VMEM_SHARED((3 * RP,), F32),
                       pltpu.SemaphoreType.DMA,
                       pltpu.SemaphoreType.DMA],
    )(_sc_gather_deg_body)


def _sc_gather_deg_body(emb_h, feat_h, pidx_h, midx_h, didx_h, z1d_h,
                        o_pe, o_pf, o_mf, o_deg,
                        pidx_v, midx_v, didx_v, rows_v, ones_v, deg_acc,
                        sem_a, sem_b):
    c = lax.axis_index("c")
    s = lax.axis_index("s")
    w1d = pl.multiple_of(s * 1920, 8)
    pltpu.sync_copy(z1d_h, deg_acc.at[pl.ds(w1d, 1920)])
    for i in range(8):
        ones_v[pl.ds(i * 16, 16)] = jnp.ones((16,), F32)
    plsc.subcore_barrier()
    pltpu.sync_copy(pidx_h.at[c, s], pidx_v)
    base_p = pl.multiple_of((c * 16 + s) * 320, 8)
    for t in range(4):
        pltpu.async_copy(emb_h.at[pidx_v.at[t]], rows_v.at[pl.ds(t * 80, 80)], sem_a)
    for t in range(4):
        pltpu.async_copy(feat_h.at[pidx_v.at[t]], rows_v.at[pl.ds(320 + t * 80, 80)], sem_b)
    for br in range(3):
        pltpu.sync_copy(didx_h.at[br, c, s], didx_v)

        def dbody(ch, carry):
            pltpu.sync_copy(ones_v, deg_acc.at[didx_v.at[ch]], add=True)
            return carry

        lax.fori_loop(0, 40, dbody, 0)
    pltpu.make_async_copy(o_pe.at[pl.ds(0, 320)], rows_v.at[pl.ds(0, 320)], sem_a).wait()
    pltpu.sync_copy(rows_v.at[pl.ds(0, 320)], o_pe.at[pl.ds(base_p, 320)])
    pltpu.make_async_copy(o_pe.at[pl.ds(0, 320)], rows_v.at[pl.ds(320, 320)], sem_b).wait()
    pltpu.sync_copy(rows_v.at[pl.ds(320, 320)], o_pf.at[pl.ds(base_p, 320)])
    pltpu.sync_copy(midx_h.at[c, s], midx_v)
    base_m = pl.multiple_of((c * 16 + s) * 640, 8)
    for t in range(8):
        pltpu.async_copy(feat_h.at[midx_v.at[t]], rows_v.at[pl.ds(t * 80, 80)], sem_a)
    pltpu.make_async_copy(o_mf.at[pl.ds(0, 640)], rows_v, sem_a).wait()
    pltpu.sync_copy(rows_v, o_mf.at[pl.ds(base_m, 640)])
    plsc.subcore_barrier()
    pltpu.sync_copy(deg_acc.at[pl.ds(w1d, 1920)], o_deg.at[c, pl.ds(w1d, 1920)])


@functools.lru_cache(maxsize=None)
def _make_agg(G):
    S = G // 2
    acc_rows = S * STRIDE
    zw = acc_rows // 16
    nch = EP * S // 16 // 128

    @functools.partial(
        pl.kernel, mesh=_mesh(),
        out_type=jax.ShapeDtypeStruct((G * STRIDE, 128), F32),
        scratch_types=[pltpu.VMEM((nch, 128), jnp.int32),
                       pltpu.VMEM((nch, 128), jnp.int32),
                       pltpu.VMEM((128, 128), F32),
                       pltpu.VMEM((128, 128), F32),
                       pltpu.VMEM((128, 128), F32),
                       pltpu.VMEM((128, 128), F32),
                       pltpu.VMEM_SHARED((acc_rows, 128), F32),
                       pltpu.SemaphoreType.DMA,
                       pltpu.SemaphoreType.DMA,
                       pltpu.SemaphoreType.DMA,
                       pltpu.SemaphoreType.DMA,
                       pltpu.SemaphoreType.DMA,
                       pltpu.SemaphoreType.DMA,
                       pltpu.SemaphoreType.DMA,
                       pltpu.SemaphoreType.DMA],
    )
    def agg(y_h, src_h, dst_h, zeros_h, z_out, src_v, dst_v,
            r0, r1, r2, r3, acc, g0, g1, g2, g3, s0, s1, s2, s3):
        c = lax.axis_index("c")
        s = lax.axis_index("s")
        bufs = (r0, r1, r2, r3)
        gsem = (g0, g1, g2, g3)
        ssem = (s0, s1, s2, s3)
        w0 = pl.multiple_of(s * zw, 8)
        pltpu.sync_copy(zeros_h.at[pl.ds(0, zw)], acc.at[pl.ds(w0, zw)])
        pltpu.sync_copy(src_h.at[c, s], src_v)
        pltpu.sync_copy(dst_h.at[c, s], dst_v)
        plsc.subcore_barrier()
        for j in range(4):
            pltpu.async_copy(y_h.at[src_v.at[j]], bufs[j], gsem[j])

        def body(i, carry):
            base = i * 4
            for j in range(4):
                ch = base + j
                pltpu.make_async_copy(y_h.at[src_v.at[ch]], bufs[j], gsem[j]).wait()
                pltpu.async_copy(bufs[j], acc.at[dst_v.at[ch]], ssem[j], add=True)
            for j in range(4):
                ch = base + j
                pltpu.make_async_copy(bufs[j], acc.at[dst_v.at[ch]], ssem[j]).wait()
                nxt = jnp.minimum(ch + 4, nch - 1)
                pltpu.async_copy(y_h.at[src_v.at[nxt]], bufs[j], gsem[j])
            return carry

        lax.fori_loop(0, nch // 4, body, 0)
        for j in range(4):
            pltpu.make_async_copy(y_h.at[src_v.at[0]], bufs[j], gsem[j]).wait()
        plsc.subcore_barrier()
        zbase = pl.multiple_of(c * acc_rows + s * zw, 8)
        pltpu.sync_copy(acc.at[pl.ds(w0, zw)], z_out.at[pl.ds(zbase, zw)])

    return agg


@functools.lru_cache(maxsize=None)
def _get_pemb_gather():
    return functools.partial(
        pl.kernel, mesh=_mesh(),
        out_type=jax.ShapeDtypeStruct((RM, 128), F32),
        scratch_types=[pltpu.VMEM((8, 80), jnp.int32),
                       pltpu.VMEM((640, 128), F32),
                       pltpu.SemaphoreType.DMA],
    )(_sc_pemb_gather_body)


def _sc_pemb_gather_body(pemb_h, idx_h, o, idx_v, rows_v, sem):
    c = lax.axis_index("c")
    s = lax.axis_index("s")
    pltpu.sync_copy(idx_h.at[c, s], idx_v)
    base = pl.multiple_of((c * 16 + s) * 640, 8)
    for t in range(8):
        pltpu.async_copy(pemb_h.at[idx_v.at[t]], rows_v.at[pl.ds(t * 80, 80)], sem)
    pltpu.make_async_copy(o.at[pl.ds(0, 640)], rows_v, sem).wait()
    pltpu.sync_copy(rows_v, o.at[pl.ds(base, 640)])


def _tc_dinv(deg_part):
    dp = deg_part.reshape(2, 3 * RP, 1)

    def body(p_ref, o_ref):
        p = p_ref[...]
        o_ref[...] = lax.rsqrt(p[0] + p[1] + 1.0)

    return pl.pallas_call(
        body,
        grid=(40,),
        in_specs=[pl.BlockSpec((2, 768, 1), lambda i: (0, i, 0))],
        out_specs=pl.BlockSpec((768, 1), lambda i: (i, 0)),
        out_shape=jax.ShapeDtypeStruct((3 * RP, 1), F32),
    )(dp)


def _tc_front(rows_e, rows_f, dinv, W1a, W1b, b1, W2, b2, gW):
    R = rows_e.shape[0]

    def body(e_ref, f_ref, dv_ref, W1a_ref, W1b_ref, b1_ref, W2_ref, b2_ref,
             gW_ref, o_ref):
        x = _mm(e_ref[...], W1a_ref[...]) + _mm(f_ref[...], W1b_ref[...]) + b1_ref[...]
        x = _mm(jnp.maximum(x, 0.0), W2_ref[...]) + b2_ref[...]
        o_ref[...] = dv_ref[...] * _mm(x, gW_ref[...])

    w = pl.BlockSpec((128, 128), lambda i: (0, 0))
    bspec = pl.BlockSpec((1, 128), lambda i: (0, 0))
    return pl.pallas_call(
        body,
        grid=(R // 256,),
        in_specs=[pl.BlockSpec((256, 128), lambda i: (i, 0)),
                  pl.BlockSpec((256, 128), lambda i: (i, 0)),
                  pl.BlockSpec((256, 1), lambda i: (i, 0)),
                  w, w, bspec, w, bspec, w],
        out_specs=pl.BlockSpec((256, 128), lambda i: (i, 0)),
        out_shape=jax.ShapeDtypeStruct((R, 128), F32),
    )(rows_e, rows_f, dinv, W1a, W1b, b1, W2, b2, gW)


def _tc_mid(z, y, dinv, b1, gW):
    R = z.shape[0]

    def body(z_ref, y_ref, dv_ref, b1_ref, gW_ref, o_ref):
        dv = dv_ref[...]
        x = jnp.maximum(dv * (z_ref[...] + y_ref[...]) + b1_ref[...], 0.0)
        o_ref[...] = dv * _mm(x, gW_ref[...])

    return pl.pallas_call(
        body,
        grid=(R // 256,),
        in_specs=[pl.BlockSpec((256, 128), lambda i: (i, 0)),
                  pl.BlockSpec((256, 128), lambda i: (i, 0)),
                  pl.BlockSpec((256, 1), lambda i: (i, 0)),
                  pl.BlockSpec((1, 128), lambda i: (0, 0)),
                  pl.BlockSpec((128, 128), lambda i: (0, 0))],
        out_specs=pl.BlockSpec((256, 128), lambda i: (i, 0)),
        out_shape=jax.ShapeDtypeStruct((R, 128), F32),
    )(z, y, dinv, b1, gW)


def _tc_tail(z, y, dinv, b2, roots, emit_emb):
    R = z.shape[0]
    G = R // STRIDE

    def body(z_ref, y_ref, dv_ref, b2_ref, root_ref, *outs):
        emb = dv_ref[...] * (z_ref[...] + y_ref[...]) + b2_ref[...]
        rid = root_ref[pl.program_id(0), 0]
        iota = lax.broadcasted_iota(jnp.int32, (STRIDE, 1), 0)
        root_row = jnp.sum(jnp.where(iota == rid, emb, 0.0), axis=0, keepdims=True)
        root_blk = jnp.broadcast_to(root_row.reshape(1, 1, 128), (1, 8, 128))
        if emit_emb:
            outs[0][...] = emb
            outs[1][...] = root_blk
        else:
            outs[0][...] = root_blk

    out_specs = [pl.BlockSpec((1, 8, 128), lambda i: (i, 0, 0))]
    out_shape = [jax.ShapeDtypeStruct((G, 8, 128), F32)]
    if emit_emb:
        out_specs = [pl.BlockSpec((STRIDE, 128), lambda i: (i, 0))] + out_specs
        out_shape = [jax.ShapeDtypeStruct((R, 128), F32)] + out_shape
    return pl.pallas_call(
        body,
        grid=(G,),
        in_specs=[pl.BlockSpec((STRIDE, 128), lambda i: (i, 0)),
                  pl.BlockSpec((STRIDE, 128), lambda i: (i, 0)),
                  pl.BlockSpec((STRIDE, 1), lambda i: (i, 0)),
                  pl.BlockSpec((1, 128), lambda i: (0, 0)),
                  pl.BlockSpec(memory_space=pltpu.SMEM)],
        out_specs=out_specs,
        out_shape=out_shape,
    )(z, y, dinv, b2, roots)


def _tc_score(pos_root, mal_roots, m_W1, m_b1, m_W2, m_b2,
              m1_W1, m1_b1, m1_W2, m1_b2):
    def body(pr, mr, W1, b1, W2, b2, V1, c1, V2, c2, o0, o1, o2, o3):
        def sc(x, A1, a1, A2, a2):
            h = jnp.maximum(_mm(x, A1[...]) + a1[...], 0.0)
            t = _mm(h, A2[...]) + a2[...]
            return 1.0 / (1.0 + jnp.exp(-t))

        p = pr[...]
        m1 = mr[pl.ds(0, 8), :]
        m2 = mr[pl.ds(8, 8), :]
        o0[...] = sc(p, W1, b1, W2, b2)
        o1[...] = sc(p, V1, c1, V2, c2)
        o2[...] = sc(m1, W1, b1, W2, b2)
        o3[...] = sc(m2, V1, c1, V2, c2)

    full = lambda shp: pl.BlockSpec(shp, lambda: tuple(0 for _ in shp))
    outs = [jax.ShapeDtypeStruct((8, 1), F32)] * 4
    return pl.pallas_call(
        body,
        in_specs=[full((8, 128)), full((16, 128)),
                  full((128, 128)), full((1, 128)), full((128, 1)), full((1, 1)),
                  full((128, 128)), full((1, 128)), full((128, 1)), full((1, 1))],
        out_specs=[full((8, 1))] * 4,
        out_shape=outs,
    )(pos_root, mal_roots, m_W1, m_b1.reshape(1, 128), m_W2, m_b2.reshape(1, 1),
      m1_W1, m1_b1.reshape(1, 128), m1_W2, m1_b2.reshape(1, 1))


def kernel(embeddings, features, pos_nodes, pos_edge_index, pos_root_local,
           mal1_nodes, mal1_edge_index, mal1_pos_map, mal1_root_local,
           mal2_nodes, mal2_edge_index, mal2_pos_map, mal2_root_local,
           fe_W1, fe_b1, fe_W2, fe_b2, g1_W, g1_b, g2_W, g2_b,
           m_W1, m_b1, m_W2, m_b2, m1_W1, m1_b1, m1_W2, m1_b2):
    i32 = jnp.int32
    pos_ei = pos_edge_index.astype(i32)
    m1_ei = mal1_edge_index.astype(i32)
    m2_ei = mal2_edge_index.astype(i32)

    pidx = _pad_nodes(pos_nodes.astype(i32)).reshape(2, 16, 4, 80)
    mal_nodes = jnp.concatenate([mal1_nodes, mal2_nodes]).astype(i32)
    midx = _pad_nodes(mal_nodes).reshape(2, 16, 8, 80)
    didx = _deg_idx([pos_ei, m1_ei, m2_ei])
    psrc, pdst = _edge_idx(pos_ei, 4)
    m1src, m1dst = _edge_idx(m1_ei, 4)
    m2src, m2dst = _edge_idx(m2_ei, 4)
    pmap = jnp.concatenate([mal1_pos_map, mal2_pos_map]).astype(i32)
    pmap = jnp.concatenate(
        [pmap, jnp.broadcast_to(pmap[:, -1:], (16, STRIDE - NS))], axis=1)
    pmap = pmap + (jnp.arange(16, dtype=i32) % 8)[:, None] * STRIDE
    pembi = pmap.reshape(2, 16, 8, 80)
    z1d = jnp.zeros((1920,), F32)
    z2d = jnp.zeros((640, 128), F32)
    pos_roots = pos_root_local.astype(i32).reshape(8, 1)
    mal_roots = jnp.concatenate(
        [mal1_root_local, mal2_root_local]).astype(i32).reshape(16, 1)
    W1a, W1b = fe_W1[:128], fe_W1[128:]
    b1 = fe_b1.reshape(1, 128)
    b2 = fe_b2.reshape(1, 128)
    g1b = g1_b.reshape(1, 128)
    g2b = g2_b.reshape(1, 128)

    pe, pf, mf, deg_part = _get_gather_deg()(embeddings, features, pidx, midx,
                                             didx, z1d)
    dinv_all = _tc_dinv(deg_part)
    dinv_pos = dinv_all[:RP]
    dinv_mal = dinv_all[RP:]

    agg8 = _make_agg(8)
    y1 = _tc_front(pe, pf, dinv_pos, W1a, W1b, b1, fe_W2, b2, g1_W)
    zz1 = agg8(y1, psrc, pdst, z2d)
    y2 = _tc_mid(zz1, y1, dinv_pos, g1b, g2_W)
    zz2 = agg8(y2, psrc, pdst, z2d)
    pos_emb, pos_root = _tc_tail(zz2, y2, dinv_pos, g2b, pos_roots, True)
    pos_root = pos_root[:, 0, :]

    pr = _get_pemb_gather()(pos_emb, pembi)
    ym1 = _tc_front(pr, mf, dinv_mal, W1a, W1b, b1, fe_W2, b2, g1_W)
    roots = []
    for half, (esrc, edst) in enumerate(((m1src, m1dst), (m2src, m2dst))):
        yh = ym1[half * RP:(half + 1) * RP]
        dvh = dinv_mal[half * RP:(half + 1) * RP]
        zh1 = agg8(yh, esrc, edst, z2d)
        yh2 = _tc_mid(zh1, yh, dvh, g1b, g2_W)
        zh2 = agg8(yh2, esrc, edst, z2d)
        (rt,) = _tc_tail(zh2, yh2, dvh, g2b, mal_roots[half * 8:(half + 1) * 8],
                         False)
        roots.append(rt[:, 0, :])
    mal_root = jnp.concatenate(roots)

    return _tc_score(pos_root, mal_root, m_W1, m_b1, m_W2, m_b2,
                     m1_W1, m1_b1, m1_W2, m1_b2)


# speedup vs baseline: 1.0706x; 1.0706x over previous
import functools

import jax
import jax.numpy as jnp
from jax import lax
from jax.experimental import pallas as pl
from jax.experimental.pallas import tpu as pltpu
from jax.experimental.pallas import tpu_sc as plsc

B = 8
NS = 1250
ES = 20000
STRIDE = 1280
RP = B * STRIDE
RM = 2 * RP
EP = 20480
F32 = jnp.float32


def _mm(a, b):
    return jnp.dot(a, b, preferred_element_type=F32)


def _pad_nodes(nodes):
    G = nodes.shape[0]
    pad = jnp.broadcast_to(nodes[:, -1:], (G, STRIDE - NS))
    return jnp.concatenate([nodes, pad], axis=1).reshape(-1)


def _edge_idx(ei, S):
    G = ei.shape[0]
    g = jnp.arange(G, dtype=jnp.int32)[:, None]
    src = jnp.concatenate([ei[:, 0, :], jnp.zeros((G, EP - ES), jnp.int32)], axis=1)
    dst = jnp.concatenate([ei[:, 1, :], jnp.full((G, EP - ES), NS, jnp.int32)], axis=1)
    srcg = src + g * STRIDE
    dstl = dst + (g % S) * STRIDE
    nch = EP * S // 16 // 128
    return srcg.reshape(2, 16, nch, 128), dstl.reshape(2, 16, nch, 128)


def _deg_idx(eis):
    parts = []
    for k, ei in enumerate(eis):
        G = ei.shape[0]
        g = jnp.arange(G, dtype=jnp.int32)[:, None]
        d = (ei[:, 1, :] + g * STRIDE).reshape(-1)
        d = jnp.concatenate([d, jnp.full((G * EP - G * ES,), NS, jnp.int32)])
        parts.append(d + k * RP)
    return jnp.stack(parts).reshape(3, 2, 16, 40, 128)


@functools.lru_cache(maxsize=None)
def _mesh():
    return plsc.VectorSubcoreMesh(core_axis_name="c", subcore_axis_name="s")


@functools.lru_cache(maxsize=None)
def _get_gather_deg():
    return functools.partial(
        pl.kernel, mesh=_mesh(),
        out_type=[jax.ShapeDtypeStruct((RP, 128), F32),
                  jax.ShapeDtypeStruct((RP, 128), F32),
                  jax.ShapeDtypeStruct((RM, 128), F32),
                  jax.ShapeDtypeStruct((2, 3 * RP), F32)],
        scratch_types=[pltpu.VMEM((4, 80), jnp.int32),
                       pltpu.VMEM((8, 80), jnp.int32),
                       pltpu.VMEM((40, 128), jnp.int32),
                       pltpu.VMEM((640, 128), F32),
                       pltpu.VMEM((128,), F32),
                       pltpu.VMEM_SHARED((3 * RP,), F32),
                       pltpu.SemaphoreType.DMA,
                       pltpu.SemaphoreType.DMA],
    )(_sc_gather_deg_body)


def _sc_gather_deg_body(emb_h, feat_h, pidx_h, midx_h, didx_h, z1d_h,
                        o_pe, o_pf, o_mf, o_deg,
                        pidx_v, midx_v, didx_v, rows_v, ones_v, deg_acc,
                        sem_a, sem_b):
    c = lax.axis_index("c")
    s = lax.axis_index("s")
    w1d = pl.multiple_of(s * 1920, 8)
    pltpu.sync_copy(z1d_h, deg_acc.at[pl.ds(w1d, 1920)])
    for i in range(8):
        ones_v[pl.ds(i * 16, 16)] = jnp.ones((16,), F32)
    plsc.subcore_barrier()
    pltpu.sync_copy(pidx_h.at[c, s], pidx_v)
    base_p = pl.multiple_of((c * 16 + s) * 320, 8)
    for t in range(4):
        pltpu.async_copy(emb_h.at[pidx_v.at[t]], rows_v.at[pl.ds(t * 80, 80)], sem_a)
    for t in range(4):
        pltpu.async_copy(feat_h.at[pidx_v.at[t]], rows_v.at[pl.ds(320 + t * 80, 80)], sem_b)
    for br in range(3):
        pltpu.sync_copy(didx_h.at[br, c, s], didx_v)

        def dbody(ch, carry):
            pltpu.sync_copy(ones_v, deg_acc.at[didx_v.at[ch]], add=True)
            return carry

        lax.fori_loop(0, 40, dbody, 0)
    pltpu.make_async_copy(o_pe.at[pl.ds(0, 320)], rows_v.at[pl.ds(0, 320)], sem_a).wait()
    pltpu.sync_copy(rows_v.at[pl.ds(0, 320)], o_pe.at[pl.ds(base_p, 320)])
    pltpu.make_async_copy(o_pe.at[pl.ds(0, 320)], rows_v.at[pl.ds(320, 320)], sem_b).wait()
    pltpu.sync_copy(rows_v.at[pl.ds(320, 320)], o_pf.at[pl.ds(base_p, 320)])
    pltpu.sync_copy(midx_h.at[c, s], midx_v)
    base_m = pl.multiple_of((c * 16 + s) * 640, 8)
    for t in range(8):
        pltpu.async_copy(feat_h.at[midx_v.at[t]], rows_v.at[pl.ds(t * 80, 80)], sem_a)
    pltpu.make_async_copy(o_mf.at[pl.ds(0, 640)], rows_v, sem_a).wait()
    pltpu.sync_copy(rows_v, o_mf.at[pl.ds(base_m, 640)])
    plsc.subcore_barrier()
    pltpu.sync_copy(deg_acc.at[pl.ds(w1d, 1920)], o_deg.at[c, pl.ds(w1d, 1920)])


@functools.lru_cache(maxsize=None)
def _make_agg(phases):
    acc_rows = 4 * STRIDE
    zw = acc_rows // 16
    nch = EP * 4 // 16 // 128

    @functools.partial(
        pl.kernel, mesh=_mesh(),
        out_type=jax.ShapeDtypeStruct((phases * RP, 128), F32),
        scratch_types=[pltpu.VMEM((nch, 128), jnp.int32),
                       pltpu.VMEM((nch, 128), jnp.int32),
                       pltpu.VMEM((128, 128), F32),
                       pltpu.VMEM_SHARED((acc_rows, 128), F32),
                       pltpu.SemaphoreType.DMA],
    )
    def agg(y_h, src_h, dst_h, zeros_h, z_out, src_v, dst_v, rows_v, acc, sem):
        c = lax.axis_index("c")
        s = lax.axis_index("s")
        w0 = pl.multiple_of(s * zw, 8)
        pltpu.sync_copy(zeros_h.at[pl.ds(0, zw)], acc.at[pl.ds(w0, zw)])
        for p in range(phases):
            pltpu.sync_copy(src_h.at[p, c, s], src_v)
            pltpu.sync_copy(dst_h.at[p, c, s], dst_v)
            plsc.subcore_barrier()

            def body(ch, carry):
                pltpu.async_copy(y_h.at[src_v.at[ch]], rows_v, sem).wait()
                pltpu.sync_copy(rows_v, acc.at[dst_v.at[ch]], add=True)
                return carry

            lax.fori_loop(0, nch, body, 0)
            plsc.subcore_barrier()
            zbase = pl.multiple_of(p * RP + c * acc_rows + s * zw, 8)
            pltpu.sync_copy(acc.at[pl.ds(w0, zw)], z_out.at[pl.ds(zbase, zw)])
            if p + 1 < phases:
                pltpu.sync_copy(zeros_h.at[pl.ds(0, zw)], acc.at[pl.ds(w0, zw)])

    return agg


@functools.lru_cache(maxsize=None)
def _get_pemb_gather():
    return functools.partial(
        pl.kernel, mesh=_mesh(),
        out_type=jax.ShapeDtypeStruct((RM, 128), F32),
        scratch_types=[pltpu.VMEM((8, 80), jnp.int32),
                       pltpu.VMEM((640, 128), F32),
                       pltpu.SemaphoreType.DMA],
    )(_sc_pemb_gather_body)


def _sc_pemb_gather_body(pemb_h, idx_h, o, idx_v, rows_v, sem):
    c = lax.axis_index("c")
    s = lax.axis_index("s")
    pltpu.sync_copy(idx_h.at[c, s], idx_v)
    base = pl.multiple_of((c * 16 + s) * 640, 8)
    for t in range(8):
        pltpu.async_copy(pemb_h.at[idx_v.at[t]], rows_v.at[pl.ds(t * 80, 80)], sem)
    pltpu.make_async_copy(o.at[pl.ds(0, 640)], rows_v, sem).wait()
    pltpu.sync_copy(rows_v, o.at[pl.ds(base, 640)])


def _dv(dp):
    return lax.rsqrt(dp[0] + dp[1] + 1.0)


def _tc_front(rows_e, rows_f, dp, W1a, W1b, b1, W2, b2, gW):
    R = rows_e.shape[0]

    def body(e_ref, f_ref, dp_ref, W1a_ref, W1b_ref, b1_ref, W2_ref, b2_ref,
             gW_ref, o_ref):
        x = _mm(e_ref[...], W1a_ref[...]) + _mm(f_ref[...], W1b_ref[...]) + b1_ref[...]
        x = _mm(jnp.maximum(x, 0.0), W2_ref[...]) + b2_ref[...]
        o_ref[...] = _dv(dp_ref[...]) * _mm(x, gW_ref[...])

    w = pl.BlockSpec((128, 128), lambda i: (0, 0))
    bspec = pl.BlockSpec((1, 128), lambda i: (0, 0))
    return pl.pallas_call(
        body,
        grid=(R // 256,),
        in_specs=[pl.BlockSpec((256, 128), lambda i: (i, 0)),
                  pl.BlockSpec((256, 128), lambda i: (i, 0)),
                  pl.BlockSpec((2, 256, 1), lambda i: (0, i, 0)),
                  w, w, bspec, w, bspec, w],
        out_specs=pl.BlockSpec((256, 128), lambda i: (i, 0)),
        out_shape=jax.ShapeDtypeStruct((R, 128), F32),
    )(rows_e, rows_f, dp, W1a, W1b, b1, W2, b2, gW)


def _tc_mid(z, y, dp, b1, gW):
    R = z.shape[0]

    def body(z_ref, y_ref, dp_ref, b1_ref, gW_ref, o_ref):
        dv = _dv(dp_ref[...])
        x = jnp.maximum(dv * (z_ref[...] + y_ref[...]) + b1_ref[...], 0.0)
        o_ref[...] = dv * _mm(x, gW_ref[...])

    return pl.pallas_call(
        body,
        grid=(R // 256,),
        in_specs=[pl.BlockSpec((256, 128), lambda i: (i, 0)),
                  pl.BlockSpec((256, 128), lambda i: (i, 0)),
                  pl.BlockSpec((2, 256, 1), lambda i: (0, i, 0)),
                  pl.BlockSpec((1, 128), lambda i: (0, 0)),
                  pl.BlockSpec((128, 128), lambda i: (0, 0))],
        out_specs=pl.BlockSpec((256, 128), lambda i: (i, 0)),
        out_shape=jax.ShapeDtypeStruct((R, 128), F32),
    )(z, y, dp, b1, gW)


def _tc_tail(z, y, dp, b2, roots, emit_emb):
    R = z.shape[0]
    G = R // STRIDE

    def body(z_ref, y_ref, dp_ref, b2_ref, root_ref, *outs):
        emb = _dv(dp_ref[...]) * (z_ref[...] + y_ref[...]) + b2_ref[...]
        rid = root_ref[pl.program_id(0), 0]
        iota = lax.broadcasted_iota(jnp.int32, (STRIDE, 1), 0)
        root_row = jnp.sum(jnp.where(iota == rid, emb, 0.0), axis=0, keepdims=True)
        root_blk = jnp.broadcast_to(root_row.reshape(1, 1, 128), (1, 8, 128))
        if emit_emb:
            outs[0][...] = emb
            outs[1][...] = root_blk
        else:
            outs[0][...] = root_blk

    out_specs = [pl.BlockSpec((1, 8, 128), lambda i: (i, 0, 0))]
    out_shape = [jax.ShapeDtypeStruct((G, 8, 128), F32)]
    if emit_emb:
        out_specs = [pl.BlockSpec((STRIDE, 128), lambda i: (i, 0))] + out_specs
        out_shape = [jax.ShapeDtypeStruct((R, 128), F32)] + out_shape
    return pl.pallas_call(
        body,
        grid=(G,),
        in_specs=[pl.BlockSpec((STRIDE, 128), lambda i: (i, 0)),
                  pl.BlockSpec((STRIDE, 128), lambda i: (i, 0)),
                  pl.BlockSpec((2, STRIDE, 1), lambda i: (0, i, 0)),
                  pl.BlockSpec((1, 128), lambda i: (0, 0)),
                  pl.BlockSpec(memory_space=pltpu.SMEM)],
        out_specs=out_specs,
        out_shape=out_shape,
    )(z, y, dp, b2, roots)


def _tc_score(pos_root, mal_roots, m_W1, m_b1, m_W2, m_b2,
              m1_W1, m1_b1, m1_W2, m1_b2):
    def body(pr, mr, W1, b1, W2, b2, V1, c1, V2, c2, o0, o1, o2, o3):
        def sc(x, A1, a1, A2, a2):
            h = jnp.maximum(_mm(x, A1[...]) + a1[...], 0.0)
            t = _mm(h, A2[...]) + a2[...]
            return 1.0 / (1.0 + jnp.exp(-t))

        p = pr[...]
        m1 = mr[pl.ds(0, 8), :]
        m2 = mr[pl.ds(8, 8), :]
        o0[...] = sc(p, W1, b1, W2, b2)
        o1[...] = sc(p, V1, c1, V2, c2)
        o2[...] = sc(m1, W1, b1, W2, b2)
        o3[...] = sc(m2, V1, c1, V2, c2)

    full = lambda shp: pl.BlockSpec(shp, lambda: tuple(0 for _ in shp))
    outs = [jax.ShapeDtypeStruct((8, 1), F32)] * 4
    return pl.pallas_call(
        body,
        in_specs=[full((8, 128)), full((16, 128)),
                  full((128, 128)), full((1, 128)), full((128, 1)), full((1, 1)),
                  full((128, 128)), full((1, 128)), full((128, 1)), full((1, 1))],
        out_specs=[full((8, 1))] * 4,
        out_shape=outs,
    )(pos_root, mal_roots, m_W1, m_b1.reshape(1, 128), m_W2, m_b2.reshape(1, 1),
      m1_W1, m1_b1.reshape(1, 128), m1_W2, m1_b2.reshape(1, 1))


def kernel(embeddings, features, pos_nodes, pos_edge_index, pos_root_local,
           mal1_nodes, mal1_edge_index, mal1_pos_map, mal1_root_local,
           mal2_nodes, mal2_edge_index, mal2_pos_map, mal2_root_local,
           fe_W1, fe_b1, fe_W2, fe_b2, g1_W, g1_b, g2_W, g2_b,
           m_W1, m_b1, m_W2, m_b2, m1_W1, m1_b1, m1_W2, m1_b2):
    i32 = jnp.int32
    pos_ei = pos_edge_index.astype(i32)
    m1_ei = mal1_edge_index.astype(i32)
    m2_ei = mal2_edge_index.astype(i32)

    pidx = _pad_nodes(pos_nodes.astype(i32)).reshape(2, 16, 4, 80)
    mal_nodes = jnp.concatenate([mal1_nodes, mal2_nodes]).astype(i32)
    midx = _pad_nodes(mal_nodes).reshape(2, 16, 8, 80)
    didx = _deg_idx([pos_ei, m1_ei, m2_ei])
    psrc, pdst = _edge_idx(pos_ei, 4)
    psrc, pdst = psrc[None], pdst[None]
    m1src, m1dst = _edge_idx(m1_ei, 4)
    m2src, m2dst = _edge_idx(m2_ei, 4)
    msrc = jnp.stack([m1src, m2src + RP])
    mdst = jnp.stack([m1dst, m2dst])
    pmap = jnp.concatenate([mal1_pos_map, mal2_pos_map]).astype(i32)
    pmap = jnp.concatenate(
        [pmap, jnp.broadcast_to(pmap[:, -1:], (16, STRIDE - NS))], axis=1)
    pmap = pmap + (jnp.arange(16, dtype=i32) % 8)[:, None] * STRIDE
    pembi = pmap.reshape(2, 16, 8, 80)
    z1d = jnp.zeros((1920,), F32)
    z2d = jnp.zeros((640, 128), F32)
    pos_roots = pos_root_local.astype(i32).reshape(8, 1)
    mal_roots = jnp.concatenate(
        [mal1_root_local, mal2_root_local]).astype(i32).reshape(16, 1)
    W1a, W1b = fe_W1[:128], fe_W1[128:]
    b1 = fe_b1.reshape(1, 128)
    b2 = fe_b2.reshape(1, 128)
    g1b = g1_b.reshape(1, 128)
    g2b = g2_b.reshape(1, 128)

    pe, pf, mf, deg_part = _get_gather_deg()(embeddings, features, pidx, midx,
                                             didx, z1d)
    dp_all = deg_part.reshape(2, 3 * RP, 1)
    dp_pos = dp_all[:, :RP]
    dp_mal = dp_all[:, RP:]

    agg1 = _make_agg(1)
    agg2 = _make_agg(2)
    y1 = _tc_front(pe, pf, dp_pos, W1a, W1b, b1, fe_W2, b2, g1_W)
    zz1 = agg1(y1, psrc, pdst, z2d)
    y2 = _tc_mid(zz1, y1, dp_pos, g1b, g2_W)
    zz2 = agg1(y2, psrc, pdst, z2d)
    pos_emb, pos_root = _tc_tail(zz2, y2, dp_pos, g2b, pos_roots, True)
    pos_root = pos_root[:, 0, :]

    pr = _get_pemb_gather()(pos_emb, pembi)
    ym1 = _tc_front(pr, mf, dp_mal, W1a, W1b, b1, fe_W2, b2, g1_W)
    zm1 = agg2(ym1, msrc, mdst, z2d)
    ym2 = _tc_mid(zm1, ym1, dp_mal, g1b, g2_W)
    zm2 = agg2(ym2, msrc, mdst, z2d)
    (mal_root,) = _tc_tail(zm2, ym2, dp_mal, g2b, mal_roots, False)
    mal_root = mal_root[:, 0, :]

    return _tc_score(pos_root, mal_root, m_W1, m_b1, m_W2, m_b2,
                     m1_W1, m1_b1, m1_W2, m1_b2)


# speedup vs baseline: 1.1370x; 1.0620x over previous
import functools

import jax
import jax.numpy as jnp
from jax import lax
from jax.experimental import pallas as pl
from jax.experimental.pallas import tpu as pltpu
from jax.experimental.pallas import tpu_sc as plsc

B = 8
NS = 1250
ES = 20000
STRIDE = 1280
RP = B * STRIDE
RM = 2 * RP
EP = 20480
F32 = jnp.float32


def _mm(a, b):
    return jnp.dot(a, b, preferred_element_type=F32)


def _pad_nodes(nodes):
    G = nodes.shape[0]
    pad = jnp.broadcast_to(nodes[:, -1:], (G, STRIDE - NS))
    return jnp.concatenate([nodes, pad], axis=1).reshape(-1)


def _edge_idx(ei, S):
    G = ei.shape[0]
    g = jnp.arange(G, dtype=jnp.int32)[:, None]
    src = jnp.concatenate([ei[:, 0, :], jnp.zeros((G, EP - ES), jnp.int32)], axis=1)
    dst = jnp.concatenate([ei[:, 1, :], jnp.full((G, EP - ES), NS, jnp.int32)], axis=1)
    srcg = src + g * STRIDE
    dstl = dst + (g % S) * STRIDE
    nch = EP * S // 16 // 128
    return srcg.reshape(2, 16, nch, 128), dstl.reshape(2, 16, nch, 128)


def _deg_idx(eis):
    parts = []
    for k, ei in enumerate(eis):
        G = ei.shape[0]
        g = jnp.arange(G, dtype=jnp.int32)[:, None]
        d = (ei[:, 1, :] + g * STRIDE).reshape(-1)
        d = jnp.concatenate([d, jnp.full((G * EP - G * ES,), NS, jnp.int32)])
        parts.append(d + k * RP)
    return jnp.stack(parts).reshape(3, 2, 16, 40, 128)


@functools.lru_cache(maxsize=None)
def _mesh():
    return plsc.VectorSubcoreMesh(core_axis_name="c", subcore_axis_name="s")


@functools.lru_cache(maxsize=None)
def _get_gather_deg():
    return functools.partial(
        pl.kernel, mesh=_mesh(),
        out_type=[jax.ShapeDtypeStruct((RP, 128), F32),
                  jax.ShapeDtypeStruct((RP, 128), F32),
                  jax.ShapeDtypeStruct((RM, 128), F32),
                  jax.ShapeDtypeStruct((2, 3 * RP), F32)],
        scratch_types=[pltpu.VMEM((4, 80), jnp.int32),
                       pltpu.VMEM((8, 80), jnp.int32),
                       pltpu.VMEM((40, 128), jnp.int32),
                       pltpu.VMEM((640, 128), F32),
                       pltpu.VMEM((128,), F32),
                       pltpu.VMEM_SHARED((3 * RP,), F32),
                       pltpu.SemaphoreType.DMA,
                       pltpu.SemaphoreType.DMA],
    )(_sc_gather_deg_body)


def _sc_gather_deg_body(emb_h, feat_h, pidx_h, midx_h, didx_h, z1d_h,
                        o_pe, o_pf, o_mf, o_deg,
                        pidx_v, midx_v, didx_v, rows_v, ones_v, deg_acc,
                        sem_a, sem_b):
    c = lax.axis_index("c")
    s = lax.axis_index("s")
    w1d = pl.multiple_of(s * 1920, 8)
    pltpu.sync_copy(z1d_h, deg_acc.at[pl.ds(w1d, 1920)])
    for i in range(8):
        ones_v[pl.ds(i * 16, 16)] = jnp.ones((16,), F32)
    plsc.subcore_barrier()
    pltpu.sync_copy(pidx_h.at[c, s], pidx_v)
    base_p = pl.multiple_of((c * 16 + s) * 320, 8)
    for t in range(4):
        pltpu.async_copy(emb_h.at[pidx_v.at[t]], rows_v.at[pl.ds(t * 80, 80)], sem_a)
    for t in range(4):
        pltpu.async_copy(feat_h.at[pidx_v.at[t]], rows_v.at[pl.ds(320 + t * 80, 80)], sem_b)
    for br in range(3):
        pltpu.sync_copy(didx_h.at[br, c, s], didx_v)

        def dbody(ch, carry):
            pltpu.sync_copy(ones_v, deg_acc.at[didx_v.at[ch]], add=True)
            return carry

        lax.fori_loop(0, 40, dbody, 0)
    pltpu.make_async_copy(o_pe.at[pl.ds(0, 320)], rows_v.at[pl.ds(0, 320)], sem_a).wait()
    pltpu.sync_copy(rows_v.at[pl.ds(0, 320)], o_pe.at[pl.ds(base_p, 320)])
    pltpu.make_async_copy(o_pe.at[pl.ds(0, 320)], rows_v.at[pl.ds(320, 320)], sem_b).wait()
    pltpu.sync_copy(rows_v.at[pl.ds(320, 320)], o_pf.at[pl.ds(base_p, 320)])
    pltpu.sync_copy(midx_h.at[c, s], midx_v)
    base_m = pl.multiple_of((c * 16 + s) * 640, 8)
    for t in range(8):
        pltpu.async_copy(feat_h.at[midx_v.at[t]], rows_v.at[pl.ds(t * 80, 80)], sem_a)
    pltpu.make_async_copy(o_mf.at[pl.ds(0, 640)], rows_v, sem_a).wait()
    pltpu.sync_copy(rows_v, o_mf.at[pl.ds(base_m, 640)])
    plsc.subcore_barrier()
    pltpu.sync_copy(deg_acc.at[pl.ds(w1d, 1920)], o_deg.at[c, pl.ds(w1d, 1920)])


@functools.lru_cache(maxsize=None)
def _make_agg(G):
    S = G // 2
    acc_rows = S * STRIDE
    zw = acc_rows // 16
    nch = EP * S // 16 // 128

    @functools.partial(
        pl.kernel, mesh=_mesh(),
        out_type=jax.ShapeDtypeStruct((G * STRIDE, 128), F32),
        scratch_types=[pltpu.VMEM((nch, 128), jnp.int32),
                       pltpu.VMEM((nch, 128), jnp.int32),
                       pltpu.VMEM((128, 128), F32),
                       pltpu.VMEM_SHARED((acc_rows, 128), F32),
                       pltpu.SemaphoreType.DMA],
    )
    def agg(y_h, src_h, dst_h, zeros_h, z_out, src_v, dst_v, rows_v, acc, sem):
        c = lax.axis_index("c")
        s = lax.axis_index("s")
        w0 = pl.multiple_of(s * zw, 8)
        pltpu.sync_copy(zeros_h.at[pl.ds(0, zw)], acc.at[pl.ds(w0, zw)])
        pltpu.sync_copy(src_h.at[c, s], src_v)
        pltpu.sync_copy(dst_h.at[c, s], dst_v)
        plsc.subcore_barrier()

        def body(ch, carry):
            pltpu.async_copy(y_h.at[src_v.at[ch]], rows_v, sem).wait()
            pltpu.sync_copy(rows_v, acc.at[dst_v.at[ch]], add=True)
            return carry

        lax.fori_loop(0, nch, body, 0)
        plsc.subcore_barrier()
        zbase = pl.multiple_of(c * acc_rows + s * zw, 8)
        pltpu.sync_copy(acc.at[pl.ds(w0, zw)], z_out.at[pl.ds(zbase, zw)])

    return agg


@functools.lru_cache(maxsize=None)
def _get_pemb_gather():
    return functools.partial(
        pl.kernel, mesh=_mesh(),
        out_type=jax.ShapeDtypeStruct((RM, 128), F32),
        scratch_types=[pltpu.VMEM((8, 80), jnp.int32),
                       pltpu.VMEM((640, 128), F32),
                       pltpu.SemaphoreType.DMA],
    )(_sc_pemb_gather_body)


def _sc_pemb_gather_body(pemb_h, idx_h, o, idx_v, rows_v, sem):
    c = lax.axis_index("c")
    s = lax.axis_index("s")
    pltpu.sync_copy(idx_h.at[c, s], idx_v)
    base = pl.multiple_of((c * 16 + s) * 640, 8)
    for t in range(8):
        pltpu.async_copy(pemb_h.at[idx_v.at[t]], rows_v.at[pl.ds(t * 80, 80)], sem)
    pltpu.make_async_copy(o.at[pl.ds(0, 640)], rows_v, sem).wait()
    pltpu.sync_copy(rows_v, o.at[pl.ds(base, 640)])


def _dv(dp):
    return lax.rsqrt(dp[0] + dp[1] + 1.0)


def _tc_front(rows_e, rows_f, dp, W1a, W1b, b1, W2, b2, gW):
    R = rows_e.shape[0]

    def body(e_ref, f_ref, dp_ref, W1a_ref, W1b_ref, b1_ref, W2_ref, b2_ref,
             gW_ref, o_ref):
        x = _mm(e_ref[...], W1a_ref[...]) + _mm(f_ref[...], W1b_ref[...]) + b1_ref[...]
        x = _mm(jnp.maximum(x, 0.0), W2_ref[...]) + b2_ref[...]
        o_ref[...] = _dv(dp_ref[...]) * _mm(x, gW_ref[...])

    w = pl.BlockSpec((128, 128), lambda i: (0, 0))
    bspec = pl.BlockSpec((1, 128), lambda i: (0, 0))
    return pl.pallas_call(
        body,
        grid=(R // 256,),
        in_specs=[pl.BlockSpec((256, 128), lambda i: (i, 0)),
                  pl.BlockSpec((256, 128), lambda i: (i, 0)),
                  pl.BlockSpec((2, 256, 1), lambda i: (0, i, 0)),
                  w, w, bspec, w, bspec, w],
        out_specs=pl.BlockSpec((256, 128), lambda i: (i, 0)),
        out_shape=jax.ShapeDtypeStruct((R, 128), F32),
    )(rows_e, rows_f, dp, W1a, W1b, b1, W2, b2, gW)


def _tc_mid(z, y, dp, b1, gW):
    R = z.shape[0]

    def body(z_ref, y_ref, dp_ref, b1_ref, gW_ref, o_ref):
        dv = _dv(dp_ref[...])
        x = jnp.maximum(dv * (z_ref[...] + y_ref[...]) + b1_ref[...], 0.0)
        o_ref[...] = dv * _mm(x, gW_ref[...])

    return pl.pallas_call(
        body,
        grid=(R // 256,),
        in_specs=[pl.BlockSpec((256, 128), lambda i: (i, 0)),
                  pl.BlockSpec((256, 128), lambda i: (i, 0)),
                  pl.BlockSpec((2, 256, 1), lambda i: (0, i, 0)),
                  pl.BlockSpec((1, 128), lambda i: (0, 0)),
                  pl.BlockSpec((128, 128), lambda i: (0, 0))],
        out_specs=pl.BlockSpec((256, 128), lambda i: (i, 0)),
        out_shape=jax.ShapeDtypeStruct((R, 128), F32),
    )(z, y, dp, b1, gW)


def _tc_tail(z, y, dp, b2, roots, emit_emb):
    R = z.shape[0]
    G = R // STRIDE

    def body(z_ref, y_ref, dp_ref, b2_ref, root_ref, *outs):
        emb = _dv(dp_ref[...]) * (z_ref[...] + y_ref[...]) + b2_ref[...]
        rid = root_ref[pl.program_id(0), 0]
        iota = lax.broadcasted_iota(jnp.int32, (STRIDE, 1), 0)
        root_row = jnp.sum(jnp.where(iota == rid, emb, 0.0), axis=0, keepdims=True)
        root_blk = jnp.broadcast_to(root_row.reshape(1, 1, 128), (1, 8, 128))
        if emit_emb:
            outs[0][...] = emb
            outs[1][...] = root_blk
        else:
            outs[0][...] = root_blk

    out_specs = [pl.BlockSpec((1, 8, 128), lambda i: (i, 0, 0))]
    out_shape = [jax.ShapeDtypeStruct((G, 8, 128), F32)]
    if emit_emb:
        out_specs = [pl.BlockSpec((STRIDE, 128), lambda i: (i, 0))] + out_specs
        out_shape = [jax.ShapeDtypeStruct((R, 128), F32)] + out_shape
    return pl.pallas_call(
        body,
        grid=(G,),
        in_specs=[pl.BlockSpec((STRIDE, 128), lambda i: (i, 0)),
                  pl.BlockSpec((STRIDE, 128), lambda i: (i, 0)),
                  pl.BlockSpec((2, STRIDE, 1), lambda i: (0, i, 0)),
                  pl.BlockSpec((1, 128), lambda i: (0, 0)),
                  pl.BlockSpec(memory_space=pltpu.SMEM)],
        out_specs=out_specs,
        out_shape=out_shape,
    )(z, y, dp, b2, roots)


def _tc_score(pos_root, mal_roots, m_W1, m_b1, m_W2, m_b2,
              m1_W1, m1_b1, m1_W2, m1_b2):
    def body(pr, mr, W1, b1, W2, b2, V1, c1, V2, c2, o0, o1, o2, o3):
        def sc(x, A1, a1, A2, a2):
            h = jnp.maximum(_mm(x, A1[...]) + a1[...], 0.0)
            t = _mm(h, A2[...]) + a2[...]
            return 1.0 / (1.0 + jnp.exp(-t))

        p = pr[...]
        m1 = mr[pl.ds(0, 8), :]
        m2 = mr[pl.ds(8, 8), :]
        o0[...] = sc(p, W1, b1, W2, b2)
        o1[...] = sc(p, V1, c1, V2, c2)
        o2[...] = sc(m1, W1, b1, W2, b2)
        o3[...] = sc(m2, V1, c1, V2, c2)

    full = lambda shp: pl.BlockSpec(shp, lambda: tuple(0 for _ in shp))
    outs = [jax.ShapeDtypeStruct((8, 1), F32)] * 4
    return pl.pallas_call(
        body,
        in_specs=[full((8, 128)), full((16, 128)),
                  full((128, 128)), full((1, 128)), full((128, 1)), full((1, 1)),
                  full((128, 128)), full((1, 128)), full((128, 1)), full((1, 1))],
        out_specs=[full((8, 1))] * 4,
        out_shape=outs,
    )(pos_root, mal_roots, m_W1, m_b1.reshape(1, 128), m_W2, m_b2.reshape(1, 1),
      m1_W1, m1_b1.reshape(1, 128), m1_W2, m1_b2.reshape(1, 1))


def kernel(embeddings, features, pos_nodes, pos_edge_index, pos_root_local,
           mal1_nodes, mal1_edge_index, mal1_pos_map, mal1_root_local,
           mal2_nodes, mal2_edge_index, mal2_pos_map, mal2_root_local,
           fe_W1, fe_b1, fe_W2, fe_b2, g1_W, g1_b, g2_W, g2_b,
           m_W1, m_b1, m_W2, m_b2, m1_W1, m1_b1, m1_W2, m1_b2):
    i32 = jnp.int32
    pos_ei = pos_edge_index.astype(i32)
    m1_ei = mal1_edge_index.astype(i32)
    m2_ei = mal2_edge_index.astype(i32)

    pidx = _pad_nodes(pos_nodes.astype(i32)).reshape(2, 16, 4, 80)
    mal_nodes = jnp.concatenate([mal1_nodes, mal2_nodes]).astype(i32)
    midx = _pad_nodes(mal_nodes).reshape(2, 16, 8, 80)
    didx = _deg_idx([pos_ei, m1_ei, m2_ei])
    psrc, pdst = _edge_idx(pos_ei, 4)
    msrc, mdst = _edge_idx(jnp.concatenate([m1_ei, m2_ei]), 8)
    pmap = jnp.concatenate([mal1_pos_map, mal2_pos_map]).astype(i32)
    pmap = jnp.concatenate(
        [pmap, jnp.broadcast_to(pmap[:, -1:], (16, STRIDE - NS))], axis=1)
    pmap = pmap + (jnp.arange(16, dtype=i32) % 8)[:, None] * STRIDE
    pembi = pmap.reshape(2, 16, 8, 80)
    z1d = jnp.zeros((1920,), F32)
    z2d = jnp.zeros((640, 128), F32)
    pos_roots = pos_root_local.astype(i32).reshape(8, 1)
    mal_roots = jnp.concatenate(
        [mal1_root_local, mal2_root_local]).astype(i32).reshape(16, 1)
    W1a, W1b = fe_W1[:128], fe_W1[128:]
    b1 = fe_b1.reshape(1, 128)
    b2 = fe_b2.reshape(1, 128)
    g1b = g1_b.reshape(1, 128)
    g2b = g2_b.reshape(1, 128)

    pe, pf, mf, deg_part = _get_gather_deg()(embeddings, features, pidx, midx,
                                             didx, z1d)
    dp_all = deg_part.reshape(2, 3 * RP, 1)
    dp_pos = dp_all[:, :RP]
    dp_mal = dp_all[:, RP:]

    agg8 = _make_agg(8)
    agg16 = _make_agg(16)
    y1 = _tc_front(pe, pf, dp_pos, W1a, W1b, b1, fe_W2, b2, g1_W)
    zz1 = agg8(y1, psrc, pdst, z2d)
    y2 = _tc_mid(zz1, y1, dp_pos, g1b, g2_W)
    zz2 = agg8(y2, psrc, pdst, z2d)
    pos_emb, pos_root = _tc_tail(zz2, y2, dp_pos, g2b, pos_roots, True)
    pos_root = pos_root[:, 0, :]

    pr = _get_pemb_gather()(pos_emb, pembi)
    ym1 = _tc_front(pr, mf, dp_mal, W1a, W1b, b1, fe_W2, b2, g1_W)
    zm1 = agg16(ym1, msrc, mdst, z2d)
    ym2 = _tc_mid(zm1, ym1, dp_mal, g1b, g2_W)
    zm2 = agg16(ym2, msrc, mdst, z2d)
    (mal_root,) = _tc_tail(zm2, ym2, dp_mal, g2b, mal_roots, False)
    mal_root = mal_root[:, 0, :]

    return _tc_score(pos_root, mal_root, m_W1, m_b1, m_W2, m_b2,
                     m1_W1, m1_b1, m1_W2, m1_b2)
